# baseline (speedup 1.0000x reference)
import functools
import math

import jax
import jax.numpy as jnp
from jax.experimental import pallas as pl
from jax.experimental.pallas import tpu as pltpu

_BLK = 65536


def _body(nblk, vocab, idx_ref, t_ref, w_ref, b_ref, out_ref,
          logits_ref, m_ref, s_ref):
    i = pl.program_id(0)

    @pl.when(i == 0)
    def _init():
        m_ref[0] = -jnp.inf
        s_ref[0] = 0.0

    @pl.when(i < nblk)
    def _phase_a():
        cols = t_ref[...]
        lane = jax.lax.broadcasted_iota(jnp.int32, cols.shape, 1)
        e_col = jnp.sum(
            jnp.where(lane == idx_ref[0] % 128, cols, 0.0),
            axis=1, keepdims=True)
        vals = jax.lax.dot_general(
            e_col, w_ref[...], (((0,), (0,)), ((), ())),
            preferred_element_type=jnp.float32)
        vals = vals + b_ref[...]
        col = jax.lax.broadcasted_iota(jnp.int32, (1, _BLK), 1) + i * _BLK
        vals = jnp.where(col < vocab, vals, -jnp.inf)
        logits_ref[pl.ds(i, 1), :] = vals
        m_old = m_ref[0]
        m_new = jnp.maximum(m_old, jnp.max(vals))
        s_ref[0] = (s_ref[0] * jnp.exp(m_old - m_new)
                    + jnp.sum(jnp.exp(vals - m_new)))
        m_ref[0] = m_new

    @pl.when(i >= nblk)
    def _phase_b():
        j = i - nblk
        c = m_ref[0] + jnp.log(s_ref[0])
        pair = jnp.concatenate(
            [logits_ref[pl.ds(2 * j, 1), :],
             logits_ref[pl.ds(2 * j + 1, 1), :]], axis=1)
        out_ref[...] = pair - c


def kernel(inputs, table, W, b):
    vocab, dim = W.shape
    nblk = math.ceil(vocab / _BLK)
    nblk_b = math.ceil(nblk / 2)
    wt = W.T
    tt = table.T
    b2 = b.reshape(1, vocab)

    grid_spec = pltpu.PrefetchScalarGridSpec(
        num_scalar_prefetch=1,
        grid=(nblk + nblk_b,),
        in_specs=[
            pl.BlockSpec((dim, 128), lambda i, idx: (0, idx[0] // 128)),
            pl.BlockSpec((dim, _BLK),
                         lambda i, idx: (0, jnp.minimum(i, nblk - 1))),
            pl.BlockSpec((1, _BLK),
                         lambda i, idx: (0, jnp.minimum(i, nblk - 1))),
        ],
        out_specs=pl.BlockSpec((1, 2 * _BLK),
                               lambda i, idx: (0, jnp.maximum(i - nblk, 0))),
        scratch_shapes=[
            pltpu.VMEM((2 * nblk_b, _BLK), jnp.float32),
            pltpu.SMEM((1,), jnp.float32),
            pltpu.SMEM((1,), jnp.float32),
        ],
    )
    return pl.pallas_call(
        functools.partial(_body, nblk, vocab),
        grid_spec=grid_spec,
        out_shape=jax.ShapeDtypeStruct((1, vocab), jnp.float32),
    )(inputs, tt, wt, b2)


# speedup vs baseline: 1.0084x; 1.0084x over previous
import functools
import math

import jax
import jax.numpy as jnp
from jax.experimental import pallas as pl
from jax.experimental.pallas import tpu as pltpu

_BLK = 65536


def _body(nblk, vocab, idx_ref, t_ref, w_ref, b_ref, out_ref,
          logits_ref, m_ref, s_ref):
    i = pl.program_id(0)

    @pl.when(i == 0)
    def _init():
        m_ref[0] = -jnp.inf
        s_ref[0] = 0.0

    @pl.when(i < nblk)
    def _phase_a():
        cols = t_ref[...]
        lane = jax.lax.broadcasted_iota(jnp.int32, cols.shape, 1)
        e_col = jnp.sum(
            jnp.where(lane == idx_ref[0] % 128, cols, 0.0),
            axis=1, keepdims=True)
        vals = jax.lax.dot_general(
            e_col, w_ref[...], (((0,), (0,)), ((), ())),
            preferred_element_type=jnp.float32)
        vals = vals + b_ref[...]
        col = jax.lax.broadcasted_iota(jnp.int32, (1, _BLK), 1) + i * _BLK
        vals = jnp.where(col < vocab, vals, -jnp.inf)
        logits_ref[pl.ds(i, 1), :] = vals
        m_old = m_ref[0]
        m_new = jnp.maximum(m_old, jnp.max(vals))
        s_ref[0] = (s_ref[0] * jnp.exp(m_old - m_new)
                    + jnp.sum(jnp.exp(vals - m_new)))
        m_ref[0] = m_new

    @pl.when(i >= nblk)
    def _phase_b():
        j = i - nblk
        c = m_ref[0] + jnp.log(s_ref[0])
        out_ref[...] = logits_ref[pl.ds(j, 1), :] - c


def kernel(inputs, table, W, b):
    vocab, dim = W.shape
    nblk = math.ceil(vocab / _BLK)
    wt = W.T
    tt = table.T
    b2 = b.reshape(1, vocab)

    grid_spec = pltpu.PrefetchScalarGridSpec(
        num_scalar_prefetch=1,
        grid=(2 * nblk,),
        in_specs=[
            pl.BlockSpec((dim, 128), lambda i, idx: (0, idx[0] // 128)),
            pl.BlockSpec((dim, _BLK),
                         lambda i, idx: (0, jnp.minimum(i, nblk - 1))),
            pl.BlockSpec((1, _BLK),
                         lambda i, idx: (0, jnp.minimum(i, nblk - 1))),
        ],
        out_specs=pl.BlockSpec((1, _BLK),
                               lambda i, idx: (0, jnp.maximum(i - nblk, 0))),
        scratch_shapes=[
            pltpu.VMEM((nblk, _BLK), jnp.float32),
            pltpu.SMEM((1,), jnp.float32),
            pltpu.SMEM((1,), jnp.float32),
        ],
    )
    return pl.pallas_call(
        functools.partial(_body, nblk, vocab),
        grid_spec=grid_spec,
        out_shape=jax.ShapeDtypeStruct((1, vocab), jnp.float32),
    )(inputs, tt, wt, b2)
